# baseline (speedup 1.0000x reference)
import functools

import jax
import jax.numpy as jnp
from jax import lax
from jax.experimental import pallas as pl
from jax.experimental.pallas import tpu as pltpu


def _fused(src_ref, w_src_ref, w_edge_ref, bias_ref, edge_ref,
           sid_ref, did_ref, out_ref, sproj_ref, acc_ref,
           *, ns, nd, fo, te, n_tiles):
    step = pl.program_id(0)

    @pl.when(step == 0)
    def _():
        sp_t = lax.dot_general(w_src_ref[...],
                               src_ref[...].astype(jnp.bfloat16),
                               (((0,), (1,)), ((), ())),
                               preferred_element_type=jnp.float32)
        sproj_ref[...] = sp_t.astype(jnp.bfloat16)

    sid = sid_ref[...]
    oh_src = (lax.broadcasted_iota(jnp.int32, (ns, te), 0)
              == sid).astype(jnp.bfloat16)
    gath_t = lax.dot_general(sproj_ref[...], oh_src,
                             (((1,), (0,)), ((), ())),
                             preferred_element_type=jnp.float32)

    ep_t = lax.dot_general(w_edge_ref[...],
                           edge_ref[...].astype(jnp.bfloat16),
                           (((0,), (1,)), ((), ())),
                           preferred_element_type=jnp.float32)

    msg_t = jnp.concatenate(
        [(gath_t + ep_t).astype(jnp.bfloat16),
         jnp.ones((8, te), jnp.bfloat16)], axis=0)

    did = did_ref[...].T
    oh_dst = (lax.broadcasted_iota(jnp.int32, (te, nd), 1)
              == did).astype(jnp.bfloat16)
    contrib = lax.dot_general(msg_t, oh_dst, (((1,), (0,)), ((), ())),
                              preferred_element_type=jnp.float32)

    @pl.when(step == 0)
    def _():
        acc_ref[...] = contrib

    @pl.when(step != 0)
    def _():
        acc_ref[...] += contrib

    @pl.when(step == n_tiles - 1)
    def _():
        acc = acc_ref[0:fo, :]
        deg = acc_ref[fo:fo + 1, :]
        norm = jnp.where(deg > 0, lax.rsqrt(deg), 0.0)
        out_t = acc * norm + bias_ref[...]
        out_ref[...] = out_t.T


def kernel(src_feats, edge_feats, src_ids, dst_ids, weights, bias,
           n_dst=2048, te=16384):
    f32 = jnp.float32
    bf16 = jnp.bfloat16
    n_src, in_feat = src_feats.shape
    n_edges = edge_feats.shape[0]
    out_feat = weights.shape[1]

    assert n_edges % te == 0
    n_tiles = n_edges // te
    m = out_feat + 8

    w = weights.astype(f32)
    w_src = w[:in_feat].astype(bf16)
    w_edge = w[in_feat:].astype(bf16)

    sid_row = src_ids.astype(jnp.int32).reshape(1, n_edges)
    did_row = dst_ids.astype(jnp.int32).reshape(1, n_edges)
    bias_col = bias.astype(f32).reshape(out_feat, 1)

    out = pl.pallas_call(
        functools.partial(_fused, ns=n_src, nd=n_dst, fo=out_feat, te=te,
                          n_tiles=n_tiles),
        grid=(n_tiles,),
        in_specs=[
            pl.BlockSpec((n_src, in_feat), lambda e: (0, 0)),
            pl.BlockSpec((in_feat, out_feat), lambda e: (0, 0)),
            pl.BlockSpec((in_feat, out_feat), lambda e: (0, 0)),
            pl.BlockSpec((out_feat, 1), lambda e: (0, 0)),
            pl.BlockSpec((te, in_feat), lambda e: (e, 0)),
            pl.BlockSpec((1, te), lambda e: (0, e)),
            pl.BlockSpec((1, te), lambda e: (0, e)),
        ],
        out_specs=pl.BlockSpec((n_dst, out_feat), lambda e: (0, 0)),
        out_shape=jax.ShapeDtypeStruct((n_dst, out_feat), f32),
        scratch_shapes=[
            pltpu.VMEM((out_feat, n_src), bf16),
            pltpu.VMEM((m, n_dst), f32),
        ],
        compiler_params=pltpu.CompilerParams(
            dimension_semantics=("arbitrary",),
            vmem_limit_bytes=100 * 1024 * 1024),
    )(src_feats, w_src, w_edge, bias_col, edge_feats, sid_row, did_row)

    return out


# speedup vs baseline: 1.0352x; 1.0352x over previous
import functools

import jax
import jax.numpy as jnp
from jax import lax
from jax.experimental import pallas as pl
from jax.experimental.pallas import tpu as pltpu


def _fused(src_ref, w_ref, bias_ref, edge_ref,
           sid_ref, did_ref, out_ref, sproj_ref, acc_ref,
           *, ns, nd, fo, fi, te, n_tiles):
    step = pl.program_id(0)

    @pl.when(step == 0)
    def _():
        sp_t = lax.dot_general(w_ref[0:fi, :].astype(jnp.bfloat16),
                               src_ref[...].astype(jnp.bfloat16),
                               (((0,), (1,)), ((), ())),
                               preferred_element_type=jnp.float32)
        sproj_ref[...] = sp_t.astype(jnp.bfloat16)

    sid = sid_ref[...]
    oh_src = (lax.broadcasted_iota(jnp.int32, (ns, te), 0)
              == sid).astype(jnp.bfloat16)
    gath_t = lax.dot_general(sproj_ref[...], oh_src,
                             (((1,), (0,)), ((), ())),
                             preferred_element_type=jnp.float32)

    ep_t = lax.dot_general(w_ref[fi:2 * fi, :].astype(jnp.bfloat16),
                           edge_ref[...].astype(jnp.bfloat16),
                           (((0,), (1,)), ((), ())),
                           preferred_element_type=jnp.float32)

    msg_t = jnp.concatenate(
        [(gath_t + ep_t).astype(jnp.bfloat16),
         jnp.ones((8, te), jnp.bfloat16)], axis=0)

    did = did_ref[...].T
    oh_dst = (lax.broadcasted_iota(jnp.int32, (te, nd), 1)
              == did).astype(jnp.bfloat16)
    contrib = lax.dot_general(msg_t, oh_dst, (((1,), (0,)), ((), ())),
                              preferred_element_type=jnp.float32)

    @pl.when(step == 0)
    def _():
        acc_ref[...] = contrib

    @pl.when(step != 0)
    def _():
        acc_ref[...] += contrib

    @pl.when(step == n_tiles - 1)
    def _():
        acc = acc_ref[0:fo, :]
        deg = acc_ref[fo:fo + 1, :]
        norm = jnp.where(deg > 0, lax.rsqrt(deg), 0.0)
        out_t = acc * norm + bias_ref[...].T
        out_ref[...] = out_t.T


def kernel(src_feats, edge_feats, src_ids, dst_ids, weights, bias,
           n_dst=2048, te=8192):
    f32 = jnp.float32
    bf16 = jnp.bfloat16
    n_src, in_feat = src_feats.shape
    n_edges = edge_feats.shape[0]
    out_feat = weights.shape[1]

    assert n_edges % te == 0
    n_tiles = n_edges // te
    m = out_feat + 8

    sid_row = src_ids.astype(jnp.int32).reshape(1, n_edges)
    did_row = dst_ids.astype(jnp.int32).reshape(1, n_edges)
    bias_row = bias.astype(f32).reshape(1, out_feat)

    out = pl.pallas_call(
        functools.partial(_fused, ns=n_src, nd=n_dst, fo=out_feat,
                          fi=in_feat, te=te, n_tiles=n_tiles),
        grid=(n_tiles,),
        in_specs=[
            pl.BlockSpec((n_src, in_feat), lambda e: (0, 0)),
            pl.BlockSpec((2 * in_feat, out_feat), lambda e: (0, 0)),
            pl.BlockSpec((1, out_feat), lambda e: (0, 0)),
            pl.BlockSpec((te, in_feat), lambda e: (e, 0)),
            pl.BlockSpec((1, te), lambda e: (0, e)),
            pl.BlockSpec((1, te), lambda e: (0, e)),
        ],
        out_specs=pl.BlockSpec((n_dst, out_feat), lambda e: (0, 0)),
        out_shape=jax.ShapeDtypeStruct((n_dst, out_feat), f32),
        scratch_shapes=[
            pltpu.VMEM((out_feat, n_src), bf16),
            pltpu.VMEM((m, n_dst), f32),
        ],
        compiler_params=pltpu.CompilerParams(
            dimension_semantics=("arbitrary",),
            vmem_limit_bytes=100 * 1024 * 1024),
    )(src_feats, weights.astype(f32), bias_row, edge_feats, sid_row, did_row)

    return out


# speedup vs baseline: 1.0387x; 1.0033x over previous
import functools

import jax
import jax.numpy as jnp
from jax import lax
from jax.experimental import pallas as pl
from jax.experimental.pallas import tpu as pltpu


def _fused(src_ref, w_ref, bias_ref, edge_ref,
           sid_ref, did_ref, out_ref, sproj_ref, acc_ref,
           *, ns, nd, fo, fi, te, n_tiles):
    step = pl.program_id(0)

    @pl.when(step == 0)
    def _():
        sp_t = lax.dot_general(w_ref[0:fi, :].astype(jnp.bfloat16),
                               src_ref[...].astype(jnp.bfloat16),
                               (((0,), (1,)), ((), ())),
                               preferred_element_type=jnp.float32)
        sproj_ref[...] = sp_t.astype(jnp.bfloat16)

    w_edge = w_ref[fi:2 * fi, :].astype(jnp.bfloat16)

    def _tile(sid, did, edge):
        oh_src = (lax.broadcasted_iota(jnp.int32, (ns, te), 0)
                  == sid).astype(jnp.bfloat16)
        gath_t = lax.dot_general(sproj_ref[...], oh_src,
                                 (((1,), (0,)), ((), ())),
                                 preferred_element_type=jnp.float32)
        ep_t = lax.dot_general(w_edge, edge.astype(jnp.bfloat16),
                               (((0,), (1,)), ((), ())),
                               preferred_element_type=jnp.float32)
        msg_t = jnp.concatenate(
            [(gath_t + ep_t).astype(jnp.bfloat16),
             jnp.ones((8, te), jnp.bfloat16)], axis=0)
        oh_dst = (lax.broadcasted_iota(jnp.int32, (te, nd), 1)
                  == did.T).astype(jnp.bfloat16)
        return lax.dot_general(msg_t, oh_dst, (((1,), (0,)), ((), ())),
                               preferred_element_type=jnp.float32)

    contrib_a = _tile(sid_ref[:, 0:te], did_ref[:, 0:te],
                      edge_ref[0:te, :])
    contrib_b = _tile(sid_ref[:, te:2 * te], did_ref[:, te:2 * te],
                      edge_ref[te:2 * te, :])
    contrib = contrib_a + contrib_b

    @pl.when(step == 0)
    def _():
        acc_ref[...] = contrib

    @pl.when(step != 0)
    def _():
        acc_ref[...] += contrib

    @pl.when(step == n_tiles - 1)
    def _():
        acc = acc_ref[0:fo, :]
        deg = acc_ref[fo:fo + 1, :]
        norm = jnp.where(deg > 0, lax.rsqrt(deg), 0.0)
        out_t = acc * norm + bias_ref[...].T
        out_ref[...] = out_t.T


def kernel(src_feats, edge_feats, src_ids, dst_ids, weights, bias,
           n_dst=2048, te=4096):
    f32 = jnp.float32
    bf16 = jnp.bfloat16
    n_src, in_feat = src_feats.shape
    n_edges = edge_feats.shape[0]
    out_feat = weights.shape[1]

    assert n_edges % (2 * te) == 0
    n_tiles = n_edges // (2 * te)
    m = out_feat + 8

    sid_row = src_ids.astype(jnp.int32).reshape(1, n_edges)
    did_row = dst_ids.astype(jnp.int32).reshape(1, n_edges)
    bias_row = bias.astype(f32).reshape(1, out_feat)

    out = pl.pallas_call(
        functools.partial(_fused, ns=n_src, nd=n_dst, fo=out_feat,
                          fi=in_feat, te=te, n_tiles=n_tiles),
        grid=(n_tiles,),
        in_specs=[
            pl.BlockSpec((n_src, in_feat), lambda e: (0, 0)),
            pl.BlockSpec((2 * in_feat, out_feat), lambda e: (0, 0)),
            pl.BlockSpec((1, out_feat), lambda e: (0, 0)),
            pl.BlockSpec((2 * te, in_feat), lambda e: (e, 0)),
            pl.BlockSpec((1, 2 * te), lambda e: (0, e)),
            pl.BlockSpec((1, 2 * te), lambda e: (0, e)),
        ],
        out_specs=pl.BlockSpec((n_dst, out_feat), lambda e: (0, 0)),
        out_shape=jax.ShapeDtypeStruct((n_dst, out_feat), f32),
        scratch_shapes=[
            pltpu.VMEM((out_feat, n_src), bf16),
            pltpu.VMEM((m, n_dst), f32),
        ],
        compiler_params=pltpu.CompilerParams(
            dimension_semantics=("arbitrary",),
            vmem_limit_bytes=100 * 1024 * 1024),
    )(src_feats, weights.astype(f32), bias_row, edge_feats, sid_row, did_row)

    return out
